# baseline (speedup 1.0000x reference)
import jax
import jax.numpy as jnp
from jax.experimental import pallas as pl
from jax.experimental.pallas import tpu as pltpu

B, S, F = 64, 25, 256
NE, NH = 10, 4
LK = 32


def _gat_body(x_ref, preds_ref, pt_ref, thr_ref, w_ref, asrc_ref, atrg_ref,
              b_ref, o_ref):
    x = x_ref[0, 0]
    w = w_ref[0]
    proj = jnp.dot(x, w, preferred_element_type=jnp.float32)

    thr = thr_ref[0, 0]
    m = preds_ref[0, 0] >= thr
    mt = pt_ref[0, 0] >= thr

    asrc2 = asrc_ref[0]
    atrg2 = atrg_ref[0]
    cs_cols, ct_cols = [], []
    for h in range(NH):
        wh = w[:, h * F:(h + 1) * F]
        cs_cols.append(jax.lax.dot_general(
            wh, asrc2[h:h + 1, :], (((1,), (1,)), ((), ())),
            preferred_element_type=jnp.float32))
        ct_cols.append(jax.lax.dot_general(
            wh, atrg2[h:h + 1, :], (((1,), (1,)), ((), ())),
            preferred_element_type=jnp.float32))
    cs = jnp.concatenate(cs_cols, axis=1)
    ct = jnp.concatenate(ct_cols, axis=1)
    ss4 = jnp.dot(x, cs, preferred_element_type=jnp.float32)
    st4 = jnp.dot(x, ct, preferred_element_type=jnp.float32)

    j_i = jax.lax.broadcasted_iota(jnp.int32, (S, NH * LK), 0)
    l_i = jax.lax.broadcasted_iota(jnp.int32, (S, NH * LK), 1)
    k_i = jnp.bitwise_and(l_i, LK - 1)
    h_i = l_i // LK
    valid = k_i < S

    ss_pack = jnp.zeros((S, NH * LK), jnp.float32)
    for h in range(NH):
        ss_pack = jnp.where(h_i == h, ss4[:, h:h + 1], ss_pack)
    st4p = jnp.concatenate(
        [st4, jnp.zeros((LK - S, NH), jnp.float32)], axis=0)
    r_i = jax.lax.broadcasted_iota(jnp.int32, (LK, NH * LK), 0)
    k_r = jnp.bitwise_and(
        jax.lax.broadcasted_iota(jnp.int32, (LK, NH * LK), 1), LK - 1)
    sel = (r_i == k_r).astype(jnp.float32)
    st_rows = jax.lax.dot_general(
        st4p, sel, (((0,), (0,)), ((), ())),
        preferred_element_type=jnp.float32)
    h_1 = jax.lax.broadcasted_iota(jnp.int32, (1, NH * LK), 1) // LK
    st_pack = jnp.zeros((1, NH * LK), jnp.float32)
    for h in range(NH):
        st_pack = jnp.where(h_1 == h, st_rows[h:h + 1, :], st_pack)

    sc = ss_pack + st_pack
    sc = jnp.where(sc >= 0, sc, 0.2 * sc)
    cmax = jnp.max(sc, axis=0, keepdims=True)
    e = jnp.exp(sc - cmax)
    base = valid & ((jnp.abs(j_i - k_i) == 1) | (j_i == k_i))
    noteye = j_i != k_i

    s_acc = jnp.zeros((S, NH * LK), jnp.float32)
    for i in range(NE):
        adj = base | (m[:, i:i + 1] & mt[i:i + 1, :] & noteye)
        den = jnp.sum(jnp.where(adj, e, 0.0), axis=0, keepdims=True)
        s_acc = s_acc + jnp.where(adj, 1.0 / den, 0.0)
    atot = e * s_acc

    acc = jnp.zeros((LK, F), jnp.float32)
    for h in range(NH):
        acc = acc + jax.lax.dot_general(
            atot[:, h * LK:(h + 1) * LK], proj[:, h * F:(h + 1) * F],
            (((0,), (0,)), ((), ())), preferred_element_type=jnp.float32)
    o_ref[0, 0] = x + b_ref[0] + acc[:S] * (1.0 / (NE * NH))


def kernel(video_features, audio_features, video_snippet_preds,
           audio_snippet_preds, confidence_threshold, aW0, a_src0, a_trg0,
           a_b0, vW0, v_src0, v_trg0, v_b0):
    xs = jnp.stack([video_features, audio_features])
    preds = jnp.stack([video_snippet_preds, audio_snippet_preds])
    pt = jnp.swapaxes(preds, 2, 3)
    pt = jnp.concatenate(
        [pt, jnp.full((2, B, NE, LK - S), -jnp.inf, jnp.float32)], axis=3)
    pt = jnp.tile(pt, (1, 1, 1, NH))
    thr = jnp.asarray(confidence_threshold, jnp.float32).reshape(1, 1)
    ws = jnp.stack([vW0, aW0])
    asrc = jnp.stack([v_src0, a_src0])
    atrg = jnp.stack([v_trg0, a_trg0])
    bs = jnp.stack([v_b0.reshape(1, F), a_b0.reshape(1, F)])

    out = pl.pallas_call(
        _gat_body,
        grid=(2, B),
        in_specs=[
            pl.BlockSpec((1, 1, S, F), lambda mo, b: (mo, b, 0, 0)),
            pl.BlockSpec((1, 1, S, NE), lambda mo, b: (mo, b, 0, 0)),
            pl.BlockSpec((1, 1, NE, NH * LK), lambda mo, b: (mo, b, 0, 0)),
            pl.BlockSpec((1, 1), lambda mo, b: (0, 0)),
            pl.BlockSpec((1, F, NH * F), lambda mo, b: (mo, 0, 0)),
            pl.BlockSpec((1, NH, F), lambda mo, b: (mo, 0, 0)),
            pl.BlockSpec((1, NH, F), lambda mo, b: (mo, 0, 0)),
            pl.BlockSpec((1, 1, F), lambda mo, b: (mo, 0, 0)),
        ],
        out_specs=pl.BlockSpec((1, 1, S, F), lambda mo, b: (mo, b, 0, 0)),
        out_shape=jax.ShapeDtypeStruct((2, B, S, F), jnp.float32),
    )(xs, preds, pt, thr, ws, asrc, atrg, bs)
    return (out[0], out[1])


# speedup vs baseline: 2.3838x; 2.3838x over previous
import jax
import jax.numpy as jnp
from jax.experimental import pallas as pl
from jax.experimental.pallas import tpu as pltpu

B, S, F = 64, 25, 256
NE, NH = 10, 4
LK = 32
SP = 32
NB = 8


def _gat_body(x_ref, preds_ref, pt_ref, thr_ref, w_ref, asrc_ref, atrg_ref,
              b_ref, o_ref, cst_ref):
    w = w_ref[0]

    @pl.when(pl.program_id(1) == 0)
    def _fold():
        cols = []
        for h in range(NH):
            wh = w[:, h * F:(h + 1) * F]
            cols.append(jax.lax.dot_general(
                wh, asrc_ref[0, h:h + 1, :], (((1,), (1,)), ((), ())),
                preferred_element_type=jnp.float32))
        for h in range(NH):
            wh = w[:, h * F:(h + 1) * F]
            cols.append(jax.lax.dot_general(
                wh, atrg_ref[0, h:h + 1, :], (((1,), (1,)), ((), ())),
                preferred_element_type=jnp.float32))
        cst_ref[...] = jnp.concatenate(cols, axis=1)

    x3 = x_ref[0]
    x2 = x3.reshape(NB * SP, F)
    proj2 = jnp.dot(x2, w, preferred_element_type=jnp.float32)
    proj3 = proj2.reshape(NB, SP, NH * F)

    sst2 = jnp.dot(x2, cst_ref[...], preferred_element_type=jnp.float32)
    sst3 = sst2.reshape(NB, SP, 2 * NH)

    thr = thr_ref[0, 0]
    m = preds_ref[0] >= thr
    mt = pt_ref[0] >= thr

    j_i = jax.lax.broadcasted_iota(jnp.int32, (NB, SP, NH * LK), 1)
    l_i = jax.lax.broadcasted_iota(jnp.int32, (NB, SP, NH * LK), 2)
    k_i = jnp.bitwise_and(l_i, LK - 1)
    h_i = l_i // LK
    valid = (k_i < S) & (j_i < S)

    ss_pack = jnp.zeros((NB, SP, NH * LK), jnp.float32)
    for h in range(NH):
        ss_pack = jnp.where(h_i == h, sst3[:, :, h:h + 1], ss_pack)
    r_i = jax.lax.broadcasted_iota(jnp.int32, (SP, NH * LK), 0)
    k_r = jnp.bitwise_and(
        jax.lax.broadcasted_iota(jnp.int32, (SP, NH * LK), 1), LK - 1)
    sel = (r_i == k_r).astype(jnp.float32)
    st_rows = jax.lax.dot_general(
        sst3[:, :, NH:], sel, (((1,), (0,)), ((), ())),
        preferred_element_type=jnp.float32)
    h_1 = jax.lax.broadcasted_iota(jnp.int32, (NB, 1, NH * LK), 2) // LK
    st_pack = jnp.zeros((NB, 1, NH * LK), jnp.float32)
    for h in range(NH):
        st_pack = jnp.where(h_1 == h, st_rows[:, h:h + 1, :], st_pack)

    sc = ss_pack + st_pack
    sc = jnp.where(sc >= 0, sc, 0.2 * sc)
    cmax = jnp.max(sc, axis=1, keepdims=True)
    e = jnp.exp(sc - cmax)

    base = valid & ((jnp.abs(j_i - k_i) == 1) | (j_i == k_i))
    noteye = j_i != k_i

    s_acc = jnp.zeros((NB, SP, NH * LK), jnp.float32)
    for i in range(NE):
        adj = base | (valid & m[:, :, i:i + 1] & mt[:, i:i + 1, :] & noteye)
        den = jnp.sum(jnp.where(adj, e, 0.0), axis=1, keepdims=True)
        s_acc = s_acc + jnp.where(adj, 1.0 / den, 0.0)
    atot = e * s_acc

    acc = x3 + b_ref[0] * jnp.float32(1.0)
    for h in range(NH):
        acc = acc + (1.0 / (NE * NH)) * jax.lax.dot_general(
            atot[:, :, h * LK:(h + 1) * LK], proj3[:, :, h * F:(h + 1) * F],
            (((1,), (1,)), ((0,), (0,))),
            preferred_element_type=jnp.float32)
    o_ref[0] = acc


def kernel(video_features, audio_features, video_snippet_preds,
           audio_snippet_preds, confidence_threshold, aW0, a_src0, a_trg0,
           a_b0, vW0, v_src0, v_trg0, v_b0):
    xs = jnp.stack([video_features, audio_features])
    xs = jnp.concatenate(
        [xs, jnp.zeros((2, B, SP - S, F), jnp.float32)], axis=2)
    preds = jnp.stack([video_snippet_preds, audio_snippet_preds])
    predsp = jnp.concatenate(
        [preds, jnp.full((2, B, SP - S, NE), -jnp.inf, jnp.float32)], axis=2)
    pt = jnp.swapaxes(preds, 2, 3)
    pt = jnp.concatenate(
        [pt, jnp.full((2, B, NE, LK - S), -jnp.inf, jnp.float32)], axis=3)
    pt = jnp.tile(pt, (1, 1, 1, NH))
    thr = jnp.asarray(confidence_threshold, jnp.float32).reshape(1, 1)
    ws = jnp.stack([vW0, aW0])
    asrc = jnp.stack([v_src0, a_src0])
    atrg = jnp.stack([v_trg0, a_trg0])
    bs = jnp.stack([v_b0.reshape(1, F), a_b0.reshape(1, F)])

    nb_blocks = B // NB
    out = pl.pallas_call(
        _gat_body,
        grid=(2, nb_blocks),
        in_specs=[
            pl.BlockSpec((1, NB, SP, F), lambda mo, b: (mo, b, 0, 0)),
            pl.BlockSpec((1, NB, SP, NE), lambda mo, b: (mo, b, 0, 0)),
            pl.BlockSpec((1, NB, NE, NH * LK), lambda mo, b: (mo, b, 0, 0)),
            pl.BlockSpec((1, 1), lambda mo, b: (0, 0)),
            pl.BlockSpec((1, F, NH * F), lambda mo, b: (mo, 0, 0)),
            pl.BlockSpec((1, NH, F), lambda mo, b: (mo, 0, 0)),
            pl.BlockSpec((1, NH, F), lambda mo, b: (mo, 0, 0)),
            pl.BlockSpec((1, 1, F), lambda mo, b: (mo, 0, 0)),
        ],
        out_specs=pl.BlockSpec((1, NB, SP, F), lambda mo, b: (mo, b, 0, 0)),
        out_shape=jax.ShapeDtypeStruct((2, B, SP, F), jnp.float32),
        scratch_shapes=[pltpu.VMEM((F, 2 * NH), jnp.float32)],
    )(xs, predsp, pt, thr, ws, asrc, atrg, bs)
    return (out[0, :, :S, :], out[1, :, :S, :])


# speedup vs baseline: 2.8939x; 1.2140x over previous
import jax
import jax.numpy as jnp
from jax.experimental import pallas as pl
from jax.experimental.pallas import tpu as pltpu

B, S, F = 64, 25, 256
NE, NH = 10, 4
LK = 32
SP = 32
NB = 8


def _gat_body(x_ref, preds_ref, pt_ref, thr_ref, w_ref, asrc_ref, atrg_ref,
              b_ref, o_ref, cst_ref):
    w = w_ref[0]

    @pl.when(pl.program_id(1) == 0)
    def _fold():
        cols = []
        for h in range(NH):
            wh = w[:, h * F:(h + 1) * F]
            cols.append(jax.lax.dot_general(
                wh, asrc_ref[0, h:h + 1, :], (((1,), (1,)), ((), ())),
                preferred_element_type=jnp.float32))
        for h in range(NH):
            wh = w[:, h * F:(h + 1) * F]
            cols.append(jax.lax.dot_general(
                wh, atrg_ref[0, h:h + 1, :], (((1,), (1,)), ((), ())),
                preferred_element_type=jnp.float32))
        cst_ref[...] = jnp.concatenate(cols, axis=1)

    x3 = x_ref[0]
    x2 = x3.reshape(NB * SP, F)
    proj2 = jnp.dot(x2, w, preferred_element_type=jnp.float32)
    proj3 = proj2.reshape(NB, SP, NH * F)

    sst2 = jnp.dot(x2, cst_ref[...], preferred_element_type=jnp.float32)
    sst3 = sst2.reshape(NB, SP, 2 * NH)

    thr = thr_ref[0, 0]
    mjf = (preds_ref[0] >= thr).astype(jnp.float32)
    mkf = (pt_ref[0] >= thr).astype(jnp.float32)

    j2 = jax.lax.broadcasted_iota(jnp.int32, (SP, NH * LK), 0)
    l2 = jax.lax.broadcasted_iota(jnp.int32, (SP, NH * LK), 1)
    k2 = jnp.bitwise_and(l2, LK - 1)
    valid2 = (k2 < S) & (j2 < S)
    base2 = valid2 & ((jnp.abs(j2 - k2) == 1) | (j2 == k2))
    base_f = base2.astype(jnp.float32)
    cl2_f = (valid2 & jnp.logical_not(base2)).astype(jnp.float32)

    hsel = (jax.lax.broadcasted_iota(jnp.int32, (NH, NH * LK), 1) // LK ==
            jax.lax.broadcasted_iota(jnp.int32, (NH, NH * LK), 0)).astype(jnp.float32)
    ss_pack = jax.lax.dot_general(
        sst3[:, :, :NH], hsel, (((2,), (0,)), ((), ())),
        preferred_element_type=jnp.float32)
    ksel = (jnp.bitwise_and(
        jax.lax.broadcasted_iota(jnp.int32, (SP, NH * LK), 1), LK - 1) ==
        jax.lax.broadcasted_iota(jnp.int32, (SP, NH * LK), 0)).astype(jnp.float32)
    st_rows = jax.lax.dot_general(
        sst3[:, :, NH:], ksel, (((1,), (0,)), ((), ())),
        preferred_element_type=jnp.float32)
    h_1 = jax.lax.broadcasted_iota(jnp.int32, (NB, 1, NH * LK), 2) // LK
    st_pack = jnp.zeros((NB, 1, NH * LK), jnp.float32)
    for h in range(NH):
        st_pack = jnp.where(h_1 == h, st_rows[:, h:h + 1, :], st_pack)

    sc = ss_pack + st_pack
    sc = jnp.where(sc >= 0, sc, 0.2 * sc)
    cmax = jnp.max(sc, axis=1, keepdims=True)
    e = jnp.exp(sc - cmax)

    g = e * cl2_f
    den_base = jnp.sum(e * base_f, axis=1, keepdims=True)
    d_cl = jax.lax.dot_general(
        mjf, g, (((1,), (1,)), ((0,), (0,))),
        preferred_element_type=jnp.float32)
    r = 1.0 / (den_base + mkf * d_cl + 1e-16)
    rk = mkf * r
    rsum = jnp.sum(r, axis=1, keepdims=True)
    s2 = jax.lax.dot_general(
        mjf, rk, (((2,), (1,)), ((0,), (0,))),
        preferred_element_type=jnp.float32)
    atot = e * (base_f * rsum + cl2_f * s2)

    acc = x3 + b_ref[0]
    for h in range(NH):
        acc = acc + (1.0 / (NE * NH)) * jax.lax.dot_general(
            atot[:, :, h * LK:(h + 1) * LK], proj3[:, :, h * F:(h + 1) * F],
            (((1,), (1,)), ((0,), (0,))),
            preferred_element_type=jnp.float32)
    o_ref[0] = acc


def kernel(video_features, audio_features, video_snippet_preds,
           audio_snippet_preds, confidence_threshold, aW0, a_src0, a_trg0,
           a_b0, vW0, v_src0, v_trg0, v_b0):
    xs = jnp.stack([video_features, audio_features])
    xs = jnp.concatenate(
        [xs, jnp.zeros((2, B, SP - S, F), jnp.float32)], axis=2)
    preds = jnp.stack([video_snippet_preds, audio_snippet_preds])
    predsp = jnp.concatenate(
        [preds, jnp.full((2, B, SP - S, NE), -jnp.inf, jnp.float32)], axis=2)
    pt = jnp.swapaxes(preds, 2, 3)
    pt = jnp.concatenate(
        [pt, jnp.full((2, B, NE, LK - S), -jnp.inf, jnp.float32)], axis=3)
    pt = jnp.tile(pt, (1, 1, 1, NH))
    thr = jnp.asarray(confidence_threshold, jnp.float32).reshape(1, 1)
    ws = jnp.stack([vW0, aW0])
    asrc = jnp.stack([v_src0, a_src0])
    atrg = jnp.stack([v_trg0, a_trg0])
    bs = jnp.stack([v_b0.reshape(1, F), a_b0.reshape(1, F)])

    nb_blocks = B // NB
    out = pl.pallas_call(
        _gat_body,
        grid=(2, nb_blocks),
        in_specs=[
            pl.BlockSpec((1, NB, SP, F), lambda mo, b: (mo, b, 0, 0)),
            pl.BlockSpec((1, NB, SP, NE), lambda mo, b: (mo, b, 0, 0)),
            pl.BlockSpec((1, NB, NE, NH * LK), lambda mo, b: (mo, b, 0, 0)),
            pl.BlockSpec((1, 1), lambda mo, b: (0, 0)),
            pl.BlockSpec((1, F, NH * F), lambda mo, b: (mo, 0, 0)),
            pl.BlockSpec((1, NH, F), lambda mo, b: (mo, 0, 0)),
            pl.BlockSpec((1, NH, F), lambda mo, b: (mo, 0, 0)),
            pl.BlockSpec((1, 1, F), lambda mo, b: (mo, 0, 0)),
        ],
        out_specs=pl.BlockSpec((1, NB, SP, F), lambda mo, b: (mo, b, 0, 0)),
        out_shape=jax.ShapeDtypeStruct((2, B, SP, F), jnp.float32),
        scratch_shapes=[pltpu.VMEM((F, 2 * NH), jnp.float32)],
    )(xs, predsp, pt, thr, ws, asrc, atrg, bs)
    return (out[0, :, :S, :], out[1, :, :S, :])


# speedup vs baseline: 3.6005x; 1.2442x over previous
import jax
import jax.numpy as jnp
from jax.experimental import pallas as pl
from jax.experimental.pallas import tpu as pltpu

B, S, F = 64, 25, 256
NE, NH = 10, 4
LK = 32
SP = 32
NB = 8


def _fold_cols(w, avec_ref):
    cols = []
    for h in range(NH):
        wh = w[:, h * F:(h + 1) * F]
        cols.append(jax.lax.dot_general(
            wh, avec_ref[h:h + 1, :], (((1,), (1,)), ((), ())),
            preferred_element_type=jnp.float32))
    return cols


def _one_modality(x3, mjf_s, w, cst, bias, hsel, ksel, base_f, cl2_f, o_ref):
    x3p = jnp.concatenate(
        [x3, jnp.zeros((NB, SP - S, F), jnp.float32)], axis=1)
    x2 = x3p.reshape(NB * SP, F)
    proj2 = jnp.dot(x2, w, preferred_element_type=jnp.float32)
    proj3 = proj2.reshape(NB, SP, NH * F)
    sst2 = jnp.dot(x2, cst, preferred_element_type=jnp.float32)
    sst3 = sst2.reshape(NB, SP, 2 * NH)

    mjf = jnp.concatenate(
        [mjf_s, jnp.zeros((NB, SP - S, NE), jnp.float32)], axis=1)
    mkf = jax.lax.dot_general(
        mjf, ksel, (((1,), (0,)), ((), ())),
        preferred_element_type=jnp.float32)

    ss_pack = jax.lax.dot_general(
        sst3[:, :, :NH], hsel, (((2,), (0,)), ((), ())),
        preferred_element_type=jnp.float32)
    st_rows = jax.lax.dot_general(
        sst3[:, :, NH:], ksel, (((1,), (0,)), ((), ())),
        preferred_element_type=jnp.float32)
    h_1 = jax.lax.broadcasted_iota(jnp.int32, (NB, 1, NH * LK), 2) // LK
    st_pack = jnp.zeros((NB, 1, NH * LK), jnp.float32)
    for h in range(NH):
        st_pack = jnp.where(h_1 == h, st_rows[:, h:h + 1, :], st_pack)

    sc = ss_pack + st_pack
    sc = jnp.where(sc >= 0, sc, 0.2 * sc)
    cmax = jnp.max(sc, axis=1, keepdims=True)
    e = jnp.exp(sc - cmax)

    g = e * cl2_f
    den_base = jnp.sum(e * base_f, axis=1, keepdims=True)
    d_cl = jax.lax.dot_general(
        mjf, g, (((1,), (1,)), ((0,), (0,))),
        preferred_element_type=jnp.float32)
    r = 1.0 / (den_base + mkf * d_cl + 1e-16)
    rk = mkf * r
    rsum = jnp.sum(r, axis=1, keepdims=True)
    s2 = jax.lax.dot_general(
        mjf, rk, (((2,), (1,)), ((0,), (0,))),
        preferred_element_type=jnp.float32)
    atot = e * (base_f * rsum + cl2_f * s2)

    acc = x3p + bias
    for h in range(NH):
        acc = acc + (1.0 / (NE * NH)) * jax.lax.dot_general(
            atot[:, :, h * LK:(h + 1) * LK], proj3[:, :, h * F:(h + 1) * F],
            (((1,), (1,)), ((0,), (0,))),
            preferred_element_type=jnp.float32)
    o_ref[...] = acc[:, :S, :]


def _gat_body(vx_ref, ax_ref, vp_ref, ap_ref, thr_ref, vw_ref, vsrc_ref,
              vtrg_ref, vb_ref, aw_ref, asrc_ref, atrg_ref, ab_ref,
              vo_ref, ao_ref, cst_ref):
    vw = vw_ref[...]
    aw = aw_ref[...]

    @pl.when(pl.program_id(0) == 0)
    def _fold():
        cols = (_fold_cols(vw, vsrc_ref) + _fold_cols(vw, vtrg_ref) +
                _fold_cols(aw, asrc_ref) + _fold_cols(aw, atrg_ref))
        cst_ref[...] = jnp.concatenate(cols, axis=1)

    j2 = jax.lax.broadcasted_iota(jnp.int32, (SP, NH * LK), 0)
    l2 = jax.lax.broadcasted_iota(jnp.int32, (SP, NH * LK), 1)
    k2 = jnp.bitwise_and(l2, LK - 1)
    valid2 = (k2 < S) & (j2 < S)
    base2 = valid2 & ((jnp.abs(j2 - k2) == 1) | (j2 == k2))
    base_f = base2.astype(jnp.float32)
    cl2_f = (valid2 & jnp.logical_not(base2)).astype(jnp.float32)
    hsel = (jax.lax.broadcasted_iota(jnp.int32, (NH, NH * LK), 1) // LK ==
            jax.lax.broadcasted_iota(jnp.int32, (NH, NH * LK), 0)).astype(jnp.float32)
    ksel = (k2 == j2).astype(jnp.float32)

    thr = thr_ref[0, 0]
    _one_modality(vx_ref[...], (vp_ref[...] >= thr).astype(jnp.float32),
                  vw, cst_ref[:, :2 * NH], vb_ref[...], hsel, ksel,
                  base_f, cl2_f, vo_ref)
    _one_modality(ax_ref[...], (ap_ref[...] >= thr).astype(jnp.float32),
                  aw, cst_ref[:, 2 * NH:], ab_ref[...], hsel, ksel,
                  base_f, cl2_f, ao_ref)


def kernel(video_features, audio_features, video_snippet_preds,
           audio_snippet_preds, confidence_threshold, aW0, a_src0, a_trg0,
           a_b0, vW0, v_src0, v_trg0, v_b0):
    thr = jnp.asarray(confidence_threshold, jnp.float32).reshape(1, 1)
    nb_blocks = B // NB
    blk = lambda b: (b, 0, 0)
    fix3 = lambda b: (0, 0, 0)
    fix2 = lambda b: (0, 0)
    vo, ao = pl.pallas_call(
        _gat_body,
        grid=(nb_blocks,),
        in_specs=[
            pl.BlockSpec((NB, S, F), blk),
            pl.BlockSpec((NB, S, F), blk),
            pl.BlockSpec((NB, S, NE), blk),
            pl.BlockSpec((NB, S, NE), blk),
            pl.BlockSpec((1, 1), fix2),
            pl.BlockSpec((F, NH * F), fix2),
            pl.BlockSpec((NH, F), fix2),
            pl.BlockSpec((NH, F), fix2),
            pl.BlockSpec((1, F), fix2),
            pl.BlockSpec((F, NH * F), fix2),
            pl.BlockSpec((NH, F), fix2),
            pl.BlockSpec((NH, F), fix2),
            pl.BlockSpec((1, F), fix2),
        ],
        out_specs=[pl.BlockSpec((NB, S, F), blk), pl.BlockSpec((NB, S, F), blk)],
        out_shape=[jax.ShapeDtypeStruct((B, S, F), jnp.float32),
                   jax.ShapeDtypeStruct((B, S, F), jnp.float32)],
        scratch_shapes=[pltpu.VMEM((F, 4 * NH), jnp.float32)],
    )(video_features, audio_features, video_snippet_preds, audio_snippet_preds,
      thr, vW0, v_src0, v_trg0, v_b0.reshape(1, F), aW0, a_src0, a_trg0,
      a_b0.reshape(1, F))
    return (vo, ao)


# speedup vs baseline: 3.7926x; 1.0534x over previous
import jax
import jax.numpy as jnp
from jax.experimental import pallas as pl
from jax.experimental.pallas import tpu as pltpu

B, S, F = 64, 25, 256
NE, NH = 10, 4
LK = 32
SP = 32
NB = 8


def _fold_cols(w, avec_ref):
    cols = []
    for h in range(NH):
        wh = w[:, h * F:(h + 1) * F]
        cols.append(jax.lax.dot_general(
            wh, avec_ref[h:h + 1, :], (((1,), (1,)), ((), ())),
            preferred_element_type=jnp.float32))
    return cols


def _one_modality(x3, mjf_s, w, cst, bias, hsel, ksel, base_f, cl2_f, o_ref):
    x3p = jnp.concatenate(
        [x3, jnp.zeros((NB, SP - S, F), jnp.float32)], axis=1)
    x2 = x3p.reshape(NB * SP, F)
    proj2 = jnp.dot(x2.astype(jnp.bfloat16), w.astype(jnp.bfloat16),
                    preferred_element_type=jnp.float32)
    proj3 = proj2.reshape(NB, SP, NH * F).astype(jnp.bfloat16)
    sst2 = jnp.dot(x2, cst, preferred_element_type=jnp.float32)
    sst3 = sst2.reshape(NB, SP, 2 * NH)

    mjf = jnp.concatenate(
        [mjf_s, jnp.zeros((NB, SP - S, NE), jnp.float32)], axis=1)
    mkf = jax.lax.dot_general(
        mjf, ksel, (((1,), (0,)), ((), ())),
        preferred_element_type=jnp.float32)

    ss_pack = jax.lax.dot_general(
        sst3[:, :, :NH], hsel, (((2,), (0,)), ((), ())),
        preferred_element_type=jnp.float32)
    st_rows = jax.lax.dot_general(
        sst3[:, :, NH:], ksel, (((1,), (0,)), ((), ())),
        preferred_element_type=jnp.float32)
    h_1 = jax.lax.broadcasted_iota(jnp.int32, (NB, 1, NH * LK), 2) // LK
    st_pack = jnp.zeros((NB, 1, NH * LK), jnp.float32)
    for h in range(NH):
        st_pack = jnp.where(h_1 == h, st_rows[:, h:h + 1, :], st_pack)

    sc = ss_pack + st_pack
    sc = jnp.where(sc >= 0, sc, 0.2 * sc)
    cmax = jnp.max(sc, axis=1, keepdims=True)
    e = jnp.exp(sc - cmax)

    g = e * cl2_f
    den_base = jnp.sum(e * base_f, axis=1, keepdims=True)
    d_cl = jax.lax.dot_general(
        mjf, g, (((1,), (1,)), ((0,), (0,))),
        preferred_element_type=jnp.float32)
    r = 1.0 / (den_base + mkf * d_cl + 1e-16)
    rk = mkf * r
    rsum = jnp.sum(r, axis=1, keepdims=True)
    s2 = jax.lax.dot_general(
        mjf, rk, (((2,), (1,)), ((0,), (0,))),
        preferred_element_type=jnp.float32)
    atot = e * (base_f * rsum + cl2_f * s2)

    acc = x3p + bias
    atot16 = atot.astype(jnp.bfloat16)
    for h in range(NH):
        acc = acc + (1.0 / (NE * NH)) * jax.lax.dot_general(
            atot16[:, :, h * LK:(h + 1) * LK], proj3[:, :, h * F:(h + 1) * F],
            (((1,), (1,)), ((0,), (0,))),
            preferred_element_type=jnp.float32)
    o_ref[...] = acc[:, :S, :]


def _gat_body(vx_ref, ax_ref, vp_ref, ap_ref, thr_ref, vw_ref, vsrc_ref,
              vtrg_ref, vb_ref, aw_ref, asrc_ref, atrg_ref, ab_ref,
              vo_ref, ao_ref, cst_ref):
    vw = vw_ref[...]
    aw = aw_ref[...]

    @pl.when(pl.program_id(0) == 0)
    def _fold():
        cols = (_fold_cols(vw, vsrc_ref) + _fold_cols(vw, vtrg_ref) +
                _fold_cols(aw, asrc_ref) + _fold_cols(aw, atrg_ref))
        cst_ref[...] = jnp.concatenate(cols, axis=1)

    j2 = jax.lax.broadcasted_iota(jnp.int32, (SP, NH * LK), 0)
    l2 = jax.lax.broadcasted_iota(jnp.int32, (SP, NH * LK), 1)
    k2 = jnp.bitwise_and(l2, LK - 1)
    valid2 = (k2 < S) & (j2 < S)
    base2 = valid2 & ((jnp.abs(j2 - k2) == 1) | (j2 == k2))
    base_f = base2.astype(jnp.float32)
    cl2_f = (valid2 & jnp.logical_not(base2)).astype(jnp.float32)
    hsel = (jax.lax.broadcasted_iota(jnp.int32, (NH, NH * LK), 1) // LK ==
            jax.lax.broadcasted_iota(jnp.int32, (NH, NH * LK), 0)).astype(jnp.float32)
    ksel = (k2 == j2).astype(jnp.float32)

    thr = thr_ref[0, 0]
    _one_modality(vx_ref[...], (vp_ref[...] >= thr).astype(jnp.float32),
                  vw, cst_ref[:, :2 * NH], vb_ref[...], hsel, ksel,
                  base_f, cl2_f, vo_ref)
    _one_modality(ax_ref[...], (ap_ref[...] >= thr).astype(jnp.float32),
                  aw, cst_ref[:, 2 * NH:], ab_ref[...], hsel, ksel,
                  base_f, cl2_f, ao_ref)


def kernel(video_features, audio_features, video_snippet_preds,
           audio_snippet_preds, confidence_threshold, aW0, a_src0, a_trg0,
           a_b0, vW0, v_src0, v_trg0, v_b0):
    thr = jnp.asarray(confidence_threshold, jnp.float32).reshape(1, 1)
    nb_blocks = B // NB
    blk = lambda b: (b, 0, 0)
    fix3 = lambda b: (0, 0, 0)
    fix2 = lambda b: (0, 0)
    vo, ao = pl.pallas_call(
        _gat_body,
        grid=(nb_blocks,),
        in_specs=[
            pl.BlockSpec((NB, S, F), blk),
            pl.BlockSpec((NB, S, F), blk),
            pl.BlockSpec((NB, S, NE), blk),
            pl.BlockSpec((NB, S, NE), blk),
            pl.BlockSpec((1, 1), fix2),
            pl.BlockSpec((F, NH * F), fix2),
            pl.BlockSpec((NH, F), fix2),
            pl.BlockSpec((NH, F), fix2),
            pl.BlockSpec((1, F), fix2),
            pl.BlockSpec((F, NH * F), fix2),
            pl.BlockSpec((NH, F), fix2),
            pl.BlockSpec((NH, F), fix2),
            pl.BlockSpec((1, F), fix2),
        ],
        out_specs=[pl.BlockSpec((NB, S, F), blk), pl.BlockSpec((NB, S, F), blk)],
        out_shape=[jax.ShapeDtypeStruct((B, S, F), jnp.float32),
                   jax.ShapeDtypeStruct((B, S, F), jnp.float32)],
        scratch_shapes=[pltpu.VMEM((F, 4 * NH), jnp.float32)],
    )(video_features, audio_features, video_snippet_preds, audio_snippet_preds,
      thr, vW0, v_src0, v_trg0, v_b0.reshape(1, F), aW0, a_src0, a_trg0,
      a_b0.reshape(1, F))
    return (vo, ao)


# speedup vs baseline: 4.2685x; 1.1255x over previous
import jax
import jax.numpy as jnp
from jax.experimental import pallas as pl
from jax.experimental.pallas import tpu as pltpu

B, S, F = 64, 25, 256
NE, NH = 10, 4
LK = 32
SP = 32
NB = 16


def _fold_cols(w, avec_ref):
    cols = []
    for h in range(NH):
        wh = w[:, h * F:(h + 1) * F]
        cols.append(jax.lax.dot_general(
            wh, avec_ref[h:h + 1, :], (((1,), (1,)), ((), ())),
            preferred_element_type=jnp.float32))
    return cols


def _one_modality(x3, mjf_s, w, cst, bias, hsel, ksel, base_f, cl2_f, o_ref):
    x3p = jnp.concatenate(
        [x3, jnp.zeros((NB, SP - S, F), jnp.float32)], axis=1)
    x2 = x3p.reshape(NB * SP, F)
    proj2 = jnp.dot(x2.astype(jnp.bfloat16), w.astype(jnp.bfloat16),
                    preferred_element_type=jnp.float32)
    proj3 = proj2.reshape(NB, SP, NH * F).astype(jnp.bfloat16)
    sst2 = jnp.dot(x2, cst, preferred_element_type=jnp.float32)
    sst3 = sst2.reshape(NB, SP, 2 * NH)

    mjf = jnp.concatenate(
        [mjf_s, jnp.zeros((NB, SP - S, NE), jnp.float32)], axis=1)
    mkf = jax.lax.dot_general(
        mjf, ksel, (((1,), (0,)), ((), ())),
        preferred_element_type=jnp.float32)

    ss_pack = jax.lax.dot_general(
        sst3[:, :, :NH], hsel, (((2,), (0,)), ((), ())),
        preferred_element_type=jnp.float32)
    st_rows = jax.lax.dot_general(
        sst3[:, :, NH:], ksel, (((1,), (0,)), ((), ())),
        preferred_element_type=jnp.float32)
    h_1 = jax.lax.broadcasted_iota(jnp.int32, (NB, 1, NH * LK), 2) // LK
    st_pack = jnp.zeros((NB, 1, NH * LK), jnp.float32)
    for h in range(NH):
        st_pack = jnp.where(h_1 == h, st_rows[:, h:h + 1, :], st_pack)

    sc = ss_pack + st_pack
    sc = jnp.where(sc >= 0, sc, 0.2 * sc)
    cmax = jnp.max(sc, axis=1, keepdims=True)
    e = jnp.exp(sc - cmax)

    g = e * cl2_f
    den_base = jnp.sum(e * base_f, axis=1, keepdims=True)
    d_cl = jax.lax.dot_general(
        mjf, g, (((1,), (1,)), ((0,), (0,))),
        preferred_element_type=jnp.float32)
    r = 1.0 / (den_base + mkf * d_cl + 1e-16)
    rk = mkf * r
    rsum = jnp.sum(r, axis=1, keepdims=True)
    s2 = jax.lax.dot_general(
        mjf, rk, (((2,), (1,)), ((0,), (0,))),
        preferred_element_type=jnp.float32)
    atot = e * (base_f * rsum + cl2_f * s2)

    acc = x3p + bias
    atot16 = atot.astype(jnp.bfloat16)
    for h in range(NH):
        acc = acc + (1.0 / (NE * NH)) * jax.lax.dot_general(
            atot16[:, :, h * LK:(h + 1) * LK], proj3[:, :, h * F:(h + 1) * F],
            (((1,), (1,)), ((0,), (0,))),
            preferred_element_type=jnp.float32)
    o_ref[...] = acc[:, :S, :]


def _gat_body(vx_ref, ax_ref, vp_ref, ap_ref, thr_ref, vw_ref, vsrc_ref,
              vtrg_ref, vb_ref, aw_ref, asrc_ref, atrg_ref, ab_ref,
              vo_ref, ao_ref, cst_ref):
    vw = vw_ref[...]
    aw = aw_ref[...]

    @pl.when(pl.program_id(0) == 0)
    def _fold():
        cols = (_fold_cols(vw, vsrc_ref) + _fold_cols(vw, vtrg_ref) +
                _fold_cols(aw, asrc_ref) + _fold_cols(aw, atrg_ref))
        cst_ref[...] = jnp.concatenate(cols, axis=1)

    j2 = jax.lax.broadcasted_iota(jnp.int32, (SP, NH * LK), 0)
    l2 = jax.lax.broadcasted_iota(jnp.int32, (SP, NH * LK), 1)
    k2 = jnp.bitwise_and(l2, LK - 1)
    valid2 = (k2 < S) & (j2 < S)
    base2 = valid2 & ((jnp.abs(j2 - k2) == 1) | (j2 == k2))
    base_f = base2.astype(jnp.float32)
    cl2_f = (valid2 & jnp.logical_not(base2)).astype(jnp.float32)
    hsel = (jax.lax.broadcasted_iota(jnp.int32, (NH, NH * LK), 1) // LK ==
            jax.lax.broadcasted_iota(jnp.int32, (NH, NH * LK), 0)).astype(jnp.float32)
    ksel = (k2 == j2).astype(jnp.float32)

    thr = thr_ref[0, 0]
    _one_modality(vx_ref[...], (vp_ref[...] >= thr).astype(jnp.float32),
                  vw, cst_ref[:, :2 * NH], vb_ref[...], hsel, ksel,
                  base_f, cl2_f, vo_ref)
    _one_modality(ax_ref[...], (ap_ref[...] >= thr).astype(jnp.float32),
                  aw, cst_ref[:, 2 * NH:], ab_ref[...], hsel, ksel,
                  base_f, cl2_f, ao_ref)


def kernel(video_features, audio_features, video_snippet_preds,
           audio_snippet_preds, confidence_threshold, aW0, a_src0, a_trg0,
           a_b0, vW0, v_src0, v_trg0, v_b0):
    thr = jnp.asarray(confidence_threshold, jnp.float32).reshape(1, 1)
    nb_blocks = B // NB
    blk = lambda b: (b, 0, 0)
    fix3 = lambda b: (0, 0, 0)
    fix2 = lambda b: (0, 0)
    vo, ao = pl.pallas_call(
        _gat_body,
        grid=(nb_blocks,),
        in_specs=[
            pl.BlockSpec((NB, S, F), blk),
            pl.BlockSpec((NB, S, F), blk),
            pl.BlockSpec((NB, S, NE), blk),
            pl.BlockSpec((NB, S, NE), blk),
            pl.BlockSpec((1, 1), fix2),
            pl.BlockSpec((F, NH * F), fix2),
            pl.BlockSpec((NH, F), fix2),
            pl.BlockSpec((NH, F), fix2),
            pl.BlockSpec((1, F), fix2),
            pl.BlockSpec((F, NH * F), fix2),
            pl.BlockSpec((NH, F), fix2),
            pl.BlockSpec((NH, F), fix2),
            pl.BlockSpec((1, F), fix2),
        ],
        out_specs=[pl.BlockSpec((NB, S, F), blk), pl.BlockSpec((NB, S, F), blk)],
        out_shape=[jax.ShapeDtypeStruct((B, S, F), jnp.float32),
                   jax.ShapeDtypeStruct((B, S, F), jnp.float32)],
        scratch_shapes=[pltpu.VMEM((F, 4 * NH), jnp.float32)],
    )(video_features, audio_features, video_snippet_preds, audio_snippet_preds,
      thr, vW0, v_src0, v_trg0, v_b0.reshape(1, F), aW0, a_src0, a_trg0,
      a_b0.reshape(1, F))
    return (vo, ao)


# speedup vs baseline: 4.6380x; 1.0866x over previous
import jax
import jax.numpy as jnp
from jax.experimental import pallas as pl
from jax.experimental.pallas import tpu as pltpu

B, S, F = 64, 25, 256
NE, NH = 10, 4
LK = 32
SP = 32
NB = 32


def _fold_cols(w, avec_ref):
    cols = []
    for h in range(NH):
        wh = w[:, h * F:(h + 1) * F]
        cols.append(jax.lax.dot_general(
            wh, avec_ref[h:h + 1, :], (((1,), (1,)), ((), ())),
            preferred_element_type=jnp.float32))
    return cols


def _one_modality(x3, mjf_s, w, cst, bias, hsel, ksel, base_f, cl2_f, o_ref):
    x3p = jnp.concatenate(
        [x3, jnp.zeros((NB, SP - S, F), jnp.float32)], axis=1)
    x2 = x3p.reshape(NB * SP, F)
    proj2 = jnp.dot(x2.astype(jnp.bfloat16), w.astype(jnp.bfloat16),
                    preferred_element_type=jnp.float32)
    proj3 = proj2.reshape(NB, SP, NH * F).astype(jnp.bfloat16)
    sst2 = jnp.dot(x2, cst, preferred_element_type=jnp.float32)
    sst3 = sst2.reshape(NB, SP, 2 * NH)

    mjf = jnp.concatenate(
        [mjf_s, jnp.zeros((NB, SP - S, NE), jnp.float32)], axis=1)
    mkf = jax.lax.dot_general(
        mjf, ksel, (((1,), (0,)), ((), ())),
        preferred_element_type=jnp.float32)

    ss_pack = jax.lax.dot_general(
        sst3[:, :, :NH], hsel, (((2,), (0,)), ((), ())),
        preferred_element_type=jnp.float32)
    st_rows = jax.lax.dot_general(
        sst3[:, :, NH:], ksel, (((1,), (0,)), ((), ())),
        preferred_element_type=jnp.float32)
    h_1 = jax.lax.broadcasted_iota(jnp.int32, (NB, 1, NH * LK), 2) // LK
    st_pack = jnp.zeros((NB, 1, NH * LK), jnp.float32)
    for h in range(NH):
        st_pack = jnp.where(h_1 == h, st_rows[:, h:h + 1, :], st_pack)

    sc = ss_pack + st_pack
    sc = jnp.where(sc >= 0, sc, 0.2 * sc)
    cmax = jnp.max(sc, axis=1, keepdims=True)
    e = jnp.exp(sc - cmax)

    g = e * cl2_f
    den_base = jnp.sum(e * base_f, axis=1, keepdims=True)
    d_cl = jax.lax.dot_general(
        mjf, g, (((1,), (1,)), ((0,), (0,))),
        preferred_element_type=jnp.float32)
    r = 1.0 / (den_base + mkf * d_cl + 1e-16)
    rk = mkf * r
    rsum = jnp.sum(r, axis=1, keepdims=True)
    s2 = jax.lax.dot_general(
        mjf, rk, (((2,), (1,)), ((0,), (0,))),
        preferred_element_type=jnp.float32)
    atot = e * (base_f * rsum + cl2_f * s2)

    acc = x3p + bias
    atot16 = atot.astype(jnp.bfloat16)
    for h in range(NH):
        acc = acc + (1.0 / (NE * NH)) * jax.lax.dot_general(
            atot16[:, :, h * LK:(h + 1) * LK], proj3[:, :, h * F:(h + 1) * F],
            (((1,), (1,)), ((0,), (0,))),
            preferred_element_type=jnp.float32)
    o_ref[...] = acc[:, :S, :]


def _gat_body(vx_ref, ax_ref, vp_ref, ap_ref, thr_ref, vw_ref, vsrc_ref,
              vtrg_ref, vb_ref, aw_ref, asrc_ref, atrg_ref, ab_ref,
              vo_ref, ao_ref, cst_ref):
    vw = vw_ref[...]
    aw = aw_ref[...]

    @pl.when(pl.program_id(0) == 0)
    def _fold():
        cols = (_fold_cols(vw, vsrc_ref) + _fold_cols(vw, vtrg_ref) +
                _fold_cols(aw, asrc_ref) + _fold_cols(aw, atrg_ref))
        cst_ref[...] = jnp.concatenate(cols, axis=1)

    j2 = jax.lax.broadcasted_iota(jnp.int32, (SP, NH * LK), 0)
    l2 = jax.lax.broadcasted_iota(jnp.int32, (SP, NH * LK), 1)
    k2 = jnp.bitwise_and(l2, LK - 1)
    valid2 = (k2 < S) & (j2 < S)
    base2 = valid2 & ((jnp.abs(j2 - k2) == 1) | (j2 == k2))
    base_f = base2.astype(jnp.float32)
    cl2_f = (valid2 & jnp.logical_not(base2)).astype(jnp.float32)
    hsel = (jax.lax.broadcasted_iota(jnp.int32, (NH, NH * LK), 1) // LK ==
            jax.lax.broadcasted_iota(jnp.int32, (NH, NH * LK), 0)).astype(jnp.float32)
    ksel = (k2 == j2).astype(jnp.float32)

    thr = thr_ref[0, 0]
    _one_modality(vx_ref[...], (vp_ref[...] >= thr).astype(jnp.float32),
                  vw, cst_ref[:, :2 * NH], vb_ref[...], hsel, ksel,
                  base_f, cl2_f, vo_ref)
    _one_modality(ax_ref[...], (ap_ref[...] >= thr).astype(jnp.float32),
                  aw, cst_ref[:, 2 * NH:], ab_ref[...], hsel, ksel,
                  base_f, cl2_f, ao_ref)


def kernel(video_features, audio_features, video_snippet_preds,
           audio_snippet_preds, confidence_threshold, aW0, a_src0, a_trg0,
           a_b0, vW0, v_src0, v_trg0, v_b0):
    thr = jnp.asarray(confidence_threshold, jnp.float32).reshape(1, 1)
    nb_blocks = B // NB
    blk = lambda b: (b, 0, 0)
    fix3 = lambda b: (0, 0, 0)
    fix2 = lambda b: (0, 0)
    vo, ao = pl.pallas_call(
        _gat_body,
        grid=(nb_blocks,),
        in_specs=[
            pl.BlockSpec((NB, S, F), blk),
            pl.BlockSpec((NB, S, F), blk),
            pl.BlockSpec((NB, S, NE), blk),
            pl.BlockSpec((NB, S, NE), blk),
            pl.BlockSpec((1, 1), fix2),
            pl.BlockSpec((F, NH * F), fix2),
            pl.BlockSpec((NH, F), fix2),
            pl.BlockSpec((NH, F), fix2),
            pl.BlockSpec((1, F), fix2),
            pl.BlockSpec((F, NH * F), fix2),
            pl.BlockSpec((NH, F), fix2),
            pl.BlockSpec((NH, F), fix2),
            pl.BlockSpec((1, F), fix2),
        ],
        out_specs=[pl.BlockSpec((NB, S, F), blk), pl.BlockSpec((NB, S, F), blk)],
        out_shape=[jax.ShapeDtypeStruct((B, S, F), jnp.float32),
                   jax.ShapeDtypeStruct((B, S, F), jnp.float32)],
        scratch_shapes=[pltpu.VMEM((F, 4 * NH), jnp.float32)],
    )(video_features, audio_features, video_snippet_preds, audio_snippet_preds,
      thr, vW0, v_src0, v_trg0, v_b0.reshape(1, F), aW0, a_src0, a_trg0,
      a_b0.reshape(1, F))
    return (vo, ao)
